# baseline (speedup 1.0000x reference)
import jax
import jax.numpy as jnp
from jax import lax
from jax.experimental import pallas as pl
from jax.experimental.pallas import tpu as pltpu

_VOCAB = 100000
_D = 64
_B = 1024

_TV = 2048
_CH = 128
_NCH = _B // _CH
_NBUF = 4


def _body(idx_ref, table_t_ref, w_t_ref, b_ref, o_ref,
          xt_vmem, buf0, buf1, buf2, buf3,
          sem0, sem1, sem2, sem3):
    bufs = (buf0, buf1, buf2, buf3)
    sems = (sem0, sem1, sem2, sem3)

    def _issue_chunk(c):
        buf, sem = bufs[c % _NBUF], sems[c % _NBUF]

        def issue(k, carry):
            v = idx_ref[c * _CH + k]
            va = pl.multiple_of((v // 128) * 128, 128)
            pltpu.make_async_copy(
                table_t_ref.at[:, pl.ds(va, 128)],
                buf.at[:, pl.ds(pl.multiple_of(k * 128, 128), 128)],
                sem,
            ).start()
            return carry

        lax.fori_loop(0, _CH, issue, 0, unroll=4)

    def _process_chunk(c):
        buf, sem = bufs[c % _NBUF], sems[c % _NBUF]
        pltpu.make_async_copy(
            table_t_ref.at[:, pl.ds(0, _CH * 128)],
            buf,
            sem,
        ).wait()

        lane = jax.lax.broadcasted_iota(jnp.int32, (_D, _CH), 1)

        def sel_k(k, acc):
            off = idx_ref[c * _CH + k] % 128
            tile = buf[:, pl.ds(pl.multiple_of(k * 128, 128), 128)]
            g = jnp.take_along_axis(
                tile, jnp.full((_D, 128), off, jnp.int32), axis=1)
            return jnp.where(lane == k, g, acc)

        sel = lax.fori_loop(0, _CH, sel_k,
                            jnp.zeros((_D, _CH), jnp.float32))
        xt_vmem[:, pl.ds(c * _CH, _CH)] = sel

    @pl.when(pl.program_id(0) == 0)
    def _gather():
        for c in range(_NBUF - 1):
            _issue_chunk(c)
        for c in range(_NCH):
            if c + _NBUF - 1 < _NCH:
                _issue_chunk(c + _NBUF - 1)
            _process_chunk(c)

    o_ref[...] = lax.dot_general(
        w_t_ref[...], xt_vmem[...],
        (((0,), (0,)), ((), ())),
        preferred_element_type=jnp.float32,
    ) + b_ref[...].T


def kernel(inputs_, emb_table, lin_w, lin_b):
    idx = inputs_.astype(jnp.int32)
    grid = pl.cdiv(_VOCAB, _TV)
    out_t = pl.pallas_call(
        _body,
        grid=(grid,),
        in_specs=[
            pl.BlockSpec(memory_space=pltpu.MemorySpace.SMEM),
            pl.BlockSpec(memory_space=pltpu.MemorySpace.HBM),
            pl.BlockSpec((_D, _TV), lambda i: (0, i)),
            pl.BlockSpec((1, _TV), lambda i: (0, i)),
        ],
        out_specs=pl.BlockSpec((_TV, _B), lambda i: (i, 0)),
        out_shape=jax.ShapeDtypeStruct((_VOCAB, _B), jnp.float32),
        scratch_shapes=[
            pltpu.VMEM((_D, _B), jnp.float32),
            pltpu.VMEM((_D, _CH * 128), jnp.float32),
            pltpu.VMEM((_D, _CH * 128), jnp.float32),
            pltpu.VMEM((_D, _CH * 128), jnp.float32),
            pltpu.VMEM((_D, _CH * 128), jnp.float32),
            pltpu.SemaphoreType.DMA,
            pltpu.SemaphoreType.DMA,
            pltpu.SemaphoreType.DMA,
            pltpu.SemaphoreType.DMA,
        ],
        compiler_params=pltpu.CompilerParams(
            dimension_semantics=("arbitrary",),
        ),
    )(idx, emb_table.T, lin_w.T, lin_b.reshape(1, _VOCAB))
    return out_t.T


# speedup vs baseline: 1.3668x; 1.3668x over previous
import jax
import jax.numpy as jnp
from jax import lax
from jax.experimental import pallas as pl
from jax.experimental.pallas import tpu as pltpu

_VOCAB = 100000
_D = 64
_B = 1024

_TV = 2048
_CH = 128
_NCH = _B // _CH
_NBUF = 4


def _body(idx_ref, table_t_ref, w_t_ref, b_ref, o_ref,
          xt_vmem, x_vmem, buf0, buf1, buf2, buf3, buf_t,
          sem0, sem1, sem2, sem3):
    bufs = (buf0, buf1, buf2, buf3)
    sems = (sem0, sem1, sem2, sem3)

    def _issue_chunk(c):
        buf, sem = bufs[c % _NBUF], sems[c % _NBUF]

        def issue(k, carry):
            v = idx_ref[c * _CH + k]
            va = pl.multiple_of((v // 128) * 128, 128)
            pltpu.make_async_copy(
                table_t_ref.at[:, pl.ds(va, 128)],
                buf.at[:, pl.ds(pl.multiple_of(k * 128, 128), 128)],
                sem,
            ).start()
            return carry

        lax.fori_loop(0, _CH, issue, 0, unroll=4)

    def _process_chunk(c):
        buf, sem = bufs[c % _NBUF], sems[c % _NBUF]
        pltpu.make_async_copy(
            table_t_ref.at[:, pl.ds(0, _CH * 128)],
            buf,
            sem,
        ).wait()

        buf_t[...] = buf[...].T

        def select(k, carry):
            j = c * _CH + k
            off = idx_ref[j] % 128
            x_vmem[pl.ds(j, 1), :] = buf_t[pl.ds(k * 128 + off, 1), :]
            return carry

        lax.fori_loop(0, _CH, select, 0, unroll=4)

    @pl.when(pl.program_id(0) == 0)
    def _gather():
        for c in range(_NBUF - 1):
            _issue_chunk(c)
        for c in range(_NCH):
            if c + _NBUF - 1 < _NCH:
                _issue_chunk(c + _NBUF - 1)
            _process_chunk(c)
        xt_vmem[...] = x_vmem[...].T

    o_ref[...] = lax.dot_general(
        w_t_ref[...], xt_vmem[...],
        (((0,), (0,)), ((), ())),
        preferred_element_type=jnp.float32,
    ) + b_ref[...].T


def kernel(inputs_, emb_table, lin_w, lin_b):
    idx = inputs_.astype(jnp.int32)
    grid = pl.cdiv(_VOCAB, _TV)
    out_t = pl.pallas_call(
        _body,
        grid=(grid,),
        in_specs=[
            pl.BlockSpec(memory_space=pltpu.MemorySpace.SMEM),
            pl.BlockSpec(memory_space=pltpu.MemorySpace.HBM),
            pl.BlockSpec((_D, _TV), lambda i: (0, i)),
            pl.BlockSpec((1, _TV), lambda i: (0, i)),
        ],
        out_specs=pl.BlockSpec((_TV, _B), lambda i: (i, 0)),
        out_shape=jax.ShapeDtypeStruct((_VOCAB, _B), jnp.float32),
        scratch_shapes=[
            pltpu.VMEM((_D, _B), jnp.float32),
            pltpu.VMEM((_B, _D), jnp.float32),
            pltpu.VMEM((_D, _CH * 128), jnp.float32),
            pltpu.VMEM((_D, _CH * 128), jnp.float32),
            pltpu.VMEM((_D, _CH * 128), jnp.float32),
            pltpu.VMEM((_D, _CH * 128), jnp.float32),
            pltpu.VMEM((_CH * 128, _D), jnp.float32),
            pltpu.SemaphoreType.DMA,
            pltpu.SemaphoreType.DMA,
            pltpu.SemaphoreType.DMA,
            pltpu.SemaphoreType.DMA,
        ],
        compiler_params=pltpu.CompilerParams(
            dimension_semantics=("arbitrary",),
        ),
    )(idx, emb_table.T, lin_w.T, lin_b.reshape(1, _VOCAB))
    return out_t.T


# speedup vs baseline: 1.4252x; 1.0427x over previous
import jax
import jax.numpy as jnp
from jax import lax
from jax.experimental import pallas as pl
from jax.experimental.pallas import tpu as pltpu

_VOCAB = 100000
_D = 64
_B = 1024

_TV = 2048
_CH = 128
_NCH = _B // _CH
_NBUF = 4


def _body(idx_ref, table_t_ref, w_t_ref, b_ref, o_ref,
          xt_vmem, buf0, buf1, buf2, buf3, sel_m,
          sem0, sem1, sem2, sem3):
    bufs = (buf0, buf1, buf2, buf3)
    sems = (sem0, sem1, sem2, sem3)

    def _issue_chunk(c):
        buf, sem = bufs[c % _NBUF], sems[c % _NBUF]

        def issue(k, carry):
            v = idx_ref[c * _CH + k]
            va = pl.multiple_of((v // 128) * 128, 128)
            pltpu.make_async_copy(
                table_t_ref.at[:, pl.ds(va, 128)],
                buf.at[:, pl.ds(pl.multiple_of(k * 128, 128), 128)],
                sem,
            ).start()
            return carry

        lax.fori_loop(0, _CH, issue, 0, unroll=4)

    def _process_chunk(c):
        buf, sem = bufs[c % _NBUF], sems[c % _NBUF]
        pltpu.make_async_copy(
            table_t_ref.at[:, pl.ds(0, _CH * 128)],
            buf,
            sem,
        ).wait()

        lane = lax.broadcasted_iota(jnp.int32, (1, _CH), 1)

        def build(k, carry):
            off = idx_ref[c * _CH + k] % 128
            sel_m[pl.ds(k * 128 + off, 1), :] = (lane == k).astype(
                jnp.float32)
            return carry

        lax.fori_loop(0, _CH, build, 0, unroll=4)

        sel = lax.dot_general(
            buf[...], sel_m[...],
            (((1,), (0,)), ((), ())),
            preferred_element_type=jnp.float32,
        )
        xt_vmem[:, pl.ds(c * _CH, _CH)] = sel

        def clear(k, carry):
            off = idx_ref[c * _CH + k] % 128
            sel_m[pl.ds(k * 128 + off, 1), :] = jnp.zeros(
                (1, _CH), jnp.float32)
            return carry

        lax.fori_loop(0, _CH, clear, 0, unroll=4)

    @pl.when(pl.program_id(0) == 0)
    def _gather():
        sel_m[...] = jnp.zeros((_CH * 128, _CH), jnp.float32)
        for c in range(_NBUF - 1):
            _issue_chunk(c)
        for c in range(_NCH):
            if c + _NBUF - 1 < _NCH:
                _issue_chunk(c + _NBUF - 1)
            _process_chunk(c)

    o_ref[...] = lax.dot_general(
        w_t_ref[...], xt_vmem[...],
        (((0,), (0,)), ((), ())),
        preferred_element_type=jnp.float32,
    ) + b_ref[...].T


def kernel(inputs_, emb_table, lin_w, lin_b):
    idx = inputs_.astype(jnp.int32)
    grid = pl.cdiv(_VOCAB, _TV)
    out_t = pl.pallas_call(
        _body,
        grid=(grid,),
        in_specs=[
            pl.BlockSpec(memory_space=pltpu.MemorySpace.SMEM),
            pl.BlockSpec(memory_space=pltpu.MemorySpace.HBM),
            pl.BlockSpec((_D, _TV), lambda i: (0, i)),
            pl.BlockSpec((1, _TV), lambda i: (0, i)),
        ],
        out_specs=pl.BlockSpec((_TV, _B), lambda i: (i, 0)),
        out_shape=jax.ShapeDtypeStruct((_VOCAB, _B), jnp.float32),
        scratch_shapes=[
            pltpu.VMEM((_D, _B), jnp.float32),
            pltpu.VMEM((_D, _CH * 128), jnp.float32),
            pltpu.VMEM((_D, _CH * 128), jnp.float32),
            pltpu.VMEM((_D, _CH * 128), jnp.float32),
            pltpu.VMEM((_D, _CH * 128), jnp.float32),
            pltpu.VMEM((_CH * 128, _CH), jnp.float32),
            pltpu.SemaphoreType.DMA,
            pltpu.SemaphoreType.DMA,
            pltpu.SemaphoreType.DMA,
            pltpu.SemaphoreType.DMA,
        ],
        compiler_params=pltpu.CompilerParams(
            dimension_semantics=("arbitrary",),
        ),
    )(idx, emb_table.T, lin_w.T, lin_b.reshape(1, _VOCAB))
    return out_t.T


# speedup vs baseline: 1.4447x; 1.0137x over previous
import jax
import jax.numpy as jnp
from jax import lax
from jax.experimental import pallas as pl
from jax.experimental.pallas import tpu as pltpu

_VOCAB = 100000
_D = 64
_B = 1024

_TV = 4096
_CH = 128
_NCH = _B // _CH
_NBUF = 2


def _body(idx_ref, table_t_ref, w_t_ref, b_ref, o_ref,
          xt_vmem, buf0, buf1, sel_m,
          sem0, sem1):
    bufs = (buf0, buf1)
    sems = (sem0, sem1)

    def _issue_chunk(c):
        buf, sem = bufs[c % _NBUF], sems[c % _NBUF]

        def issue(k, carry):
            v = idx_ref[c * _CH + k]
            va = pl.multiple_of((v // 128) * 128, 128)
            pltpu.make_async_copy(
                table_t_ref.at[:, pl.ds(va, 128)],
                buf.at[:, pl.ds(pl.multiple_of(k * 128, 128), 128)],
                sem,
            ).start()
            return carry

        lax.fori_loop(0, _CH, issue, 0, unroll=4)

    def _process_chunk(c):
        buf, sem = bufs[c % _NBUF], sems[c % _NBUF]
        pltpu.make_async_copy(
            table_t_ref.at[:, pl.ds(0, _CH * 128)],
            buf,
            sem,
        ).wait()

        lane = lax.broadcasted_iota(jnp.int32, (1, _CH), 1)

        def build(k, carry):
            off = idx_ref[c * _CH + k] % 128
            sel_m[pl.ds(k * 128 + off, 1), :] = (lane == k).astype(
                jnp.float32)
            return carry

        lax.fori_loop(0, _CH, build, 0, unroll=4)

        sel = lax.dot_general(
            buf[...], sel_m[...],
            (((1,), (0,)), ((), ())),
            preferred_element_type=jnp.float32,
        )
        xt_vmem[:, pl.ds(c * _CH, _CH)] = sel

        def clear(k, carry):
            off = idx_ref[c * _CH + k] % 128
            sel_m[pl.ds(k * 128 + off, 1), :] = jnp.zeros(
                (1, _CH), jnp.float32)
            return carry

        lax.fori_loop(0, _CH, clear, 0, unroll=4)

    @pl.when(pl.program_id(0) == 0)
    def _gather():
        sel_m[...] = jnp.zeros((_CH * 128, _CH), jnp.float32)
        for c in range(_NBUF - 1):
            _issue_chunk(c)
        for c in range(_NCH):
            if c + _NBUF - 1 < _NCH:
                _issue_chunk(c + _NBUF - 1)
            _process_chunk(c)

    o_ref[...] = lax.dot_general(
        w_t_ref[...], xt_vmem[...],
        (((0,), (0,)), ((), ())),
        preferred_element_type=jnp.float32,
    ) + b_ref[...].T


def kernel(inputs_, emb_table, lin_w, lin_b):
    idx = inputs_.astype(jnp.int32)
    grid = pl.cdiv(_VOCAB, _TV)
    out_t = pl.pallas_call(
        _body,
        grid=(grid,),
        in_specs=[
            pl.BlockSpec(memory_space=pltpu.MemorySpace.SMEM),
            pl.BlockSpec(memory_space=pltpu.MemorySpace.HBM),
            pl.BlockSpec((_D, _TV), lambda i: (0, i)),
            pl.BlockSpec((1, _TV), lambda i: (0, i)),
        ],
        out_specs=pl.BlockSpec((_TV, _B), lambda i: (i, 0)),
        out_shape=jax.ShapeDtypeStruct((_VOCAB, _B), jnp.float32),
        scratch_shapes=[
            pltpu.VMEM((_D, _B), jnp.float32),
            pltpu.VMEM((_D, _CH * 128), jnp.float32),
            pltpu.VMEM((_D, _CH * 128), jnp.float32),
            pltpu.VMEM((_CH * 128, _CH), jnp.float32),
            pltpu.SemaphoreType.DMA,
            pltpu.SemaphoreType.DMA,
        ],
        compiler_params=pltpu.CompilerParams(
            dimension_semantics=("arbitrary",),
            vmem_limit_bytes=100 * 1024 * 1024,
        ),
    )(idx, emb_table.T, lin_w.T, lin_b.reshape(1, _VOCAB))
    return out_t.T


# speedup vs baseline: 1.4493x; 1.0031x over previous
import jax
import jax.numpy as jnp
from jax import lax
from jax.experimental import pallas as pl
from jax.experimental.pallas import tpu as pltpu

_VOCAB = 100000
_D = 64
_B = 1024

_TV = 3072
_CH = 128
_NCH = _B // _CH
_NBUF = 3


def _body(idx_ref, table_t_ref, w_t_ref, b_ref, o_ref,
          xt_vmem, buf0, buf1, buf2, sel_m,
          sem0, sem1, sem2):
    bufs = (buf0, buf1, buf2)
    sems = (sem0, sem1, sem2)

    def _issue_chunk(c):
        buf, sem = bufs[c % _NBUF], sems[c % _NBUF]

        def issue(k, carry):
            v = idx_ref[c * _CH + k]
            va = pl.multiple_of((v // 128) * 128, 128)
            pltpu.make_async_copy(
                table_t_ref.at[:, pl.ds(va, 128)],
                buf.at[:, pl.ds(pl.multiple_of(k * 128, 128), 128)],
                sem,
            ).start()
            return carry

        lax.fori_loop(0, _CH, issue, 0, unroll=4)

    def _process_chunk(c):
        buf, sem = bufs[c % _NBUF], sems[c % _NBUF]
        pltpu.make_async_copy(
            table_t_ref.at[:, pl.ds(0, _CH * 128)],
            buf,
            sem,
        ).wait()

        lane = lax.broadcasted_iota(jnp.int32, (1, _CH), 1)

        def build(k, carry):
            off = idx_ref[c * _CH + k] % 128
            sel_m[pl.ds(k * 128 + off, 1), :] = (lane == k).astype(
                jnp.float32)
            return carry

        lax.fori_loop(0, _CH, build, 0, unroll=4)

        sel = lax.dot_general(
            buf[...], sel_m[...],
            (((1,), (0,)), ((), ())),
            preferred_element_type=jnp.float32,
        )
        xt_vmem[:, pl.ds(c * _CH, _CH)] = sel

        def clear(k, carry):
            off = idx_ref[c * _CH + k] % 128
            sel_m[pl.ds(k * 128 + off, 1), :] = jnp.zeros(
                (1, _CH), jnp.float32)
            return carry

        lax.fori_loop(0, _CH, clear, 0, unroll=4)

    @pl.when(pl.program_id(0) == 0)
    def _gather():
        sel_m[...] = jnp.zeros((_CH * 128, _CH), jnp.float32)
        for c in range(_NBUF - 1):
            _issue_chunk(c)
        for c in range(_NCH):
            if c + _NBUF - 1 < _NCH:
                _issue_chunk(c + _NBUF - 1)
            _process_chunk(c)

    o_ref[...] = lax.dot_general(
        w_t_ref[...], xt_vmem[...],
        (((0,), (0,)), ((), ())),
        preferred_element_type=jnp.float32,
    ) + b_ref[...].T


def kernel(inputs_, emb_table, lin_w, lin_b):
    idx = inputs_.astype(jnp.int32)
    grid = pl.cdiv(_VOCAB, _TV)
    out_t = pl.pallas_call(
        _body,
        grid=(grid,),
        in_specs=[
            pl.BlockSpec(memory_space=pltpu.MemorySpace.SMEM),
            pl.BlockSpec(memory_space=pltpu.MemorySpace.HBM),
            pl.BlockSpec((_D, _TV), lambda i: (0, i)),
            pl.BlockSpec((1, _TV), lambda i: (0, i)),
        ],
        out_specs=pl.BlockSpec((_TV, _B), lambda i: (i, 0)),
        out_shape=jax.ShapeDtypeStruct((_VOCAB, _B), jnp.float32),
        scratch_shapes=[
            pltpu.VMEM((_D, _B), jnp.float32),
            pltpu.VMEM((_D, _CH * 128), jnp.float32),
            pltpu.VMEM((_D, _CH * 128), jnp.float32),
            pltpu.VMEM((_D, _CH * 128), jnp.float32),
            pltpu.VMEM((_CH * 128, _CH), jnp.float32),
            pltpu.SemaphoreType.DMA,
            pltpu.SemaphoreType.DMA,
            pltpu.SemaphoreType.DMA,
        ],
        compiler_params=pltpu.CompilerParams(
            dimension_semantics=("arbitrary",),
            vmem_limit_bytes=100 * 1024 * 1024,
        ),
    )(idx, emb_table.T, lin_w.T, lin_b.reshape(1, _VOCAB))
    return out_t.T
